# baseline (speedup 1.0000x reference)
import functools

import jax
import jax.numpy as jnp
from jax import lax
from jax.experimental import pallas as pl
from jax.experimental.pallas import tpu as pltpu
from jax.experimental.pallas import tpu_sc as plsc

NC = 2
NS = 16
CHUNK = 80
LANES = 16
NBUF = 5
GLAG = 2


def _sc_spmm(x2, row3, col3, ew3, dh):
    n_rows = x2.shape[0]
    n_nodes = n_rows // NC
    n_chunks = row3.shape[1]
    dh_vecs = dh // LANES
    n_zblk = n_nodes // CHUNK
    zrounds = -(-n_zblk // NS)

    mesh = plsc.VectorSubcoreMesh(
        core_axis_name="c", subcore_axis_name="s", num_cores=NC, num_subcores=NS
    )

    @functools.partial(
        pl.kernel,
        out_type=jax.ShapeDtypeStruct((NC, n_nodes, dh), jnp.float32),
        mesh=mesh,
        compiler_params=pltpu.CompilerParams(use_tc_tiling_on_sc=False),
        scratch_types=[
            pltpu.VMEM((n_chunks, CHUNK), jnp.int32),
            pltpu.VMEM((n_chunks, CHUNK), jnp.int32),
            [pltpu.VMEM((CHUNK,), jnp.float32) for _ in range(NBUF)],
            [pltpu.VMEM((CHUNK, dh // 2), jnp.int32) for _ in range(NBUF)],
            [pltpu.VMEM((CHUNK, dh), jnp.float32) for _ in range(NBUF)],
            [pltpu.SemaphoreType.DMA for _ in range(NBUF)],
            [pltpu.SemaphoreType.DMA for _ in range(NBUF)],
            [pltpu.SemaphoreType.DMA for _ in range(NBUF)],
            pltpu.VMEM_SHARED((n_nodes, dh), jnp.float32),
        ],
    )
    def spmm_kernel(x_hbm, row_hbm, col_hbm, ew_hbm, out_hbm,
                    col_v, row_v, ewb, gbufs, sbufs, semg, semw, sems, acc_sh):
        cid = lax.axis_index("c")
        sid = lax.axis_index("s")

        pltpu.sync_copy(col_hbm.at[sid], col_v)
        pltpu.sync_copy(row_hbm.at[sid], row_v)

        cidv = jnp.full((LANES,), cid, jnp.int32)

        def bias_body(j, carry):
            for g in range(CHUNK // LANES):
                sl = pl.ds(g * LANES, LANES)
                col_v[j, sl] = col_v[j, sl] + col_v[j, sl] + cidv
            return carry

        lax.fori_loop(0, n_chunks, bias_body, 0)

        def zero_row(i, carry):
            for k in range(dh_vecs):
                sbufs[0][i, pl.ds(k * LANES, LANES)] = jnp.zeros((LANES,), jnp.float32)
            return carry

        lax.fori_loop(0, CHUNK, zero_row, 0)
        for t in range(zrounds):
            blk_id = sid + t * NS

            @pl.when(blk_id < n_zblk)
            def _():
                pltpu.sync_copy(sbufs[0], acc_sh.at[pl.ds(blk_id * CHUNK, CHUNK)])
        plsc.subcore_barrier()

        def issue_gather(c, b):
            pltpu.async_copy(x_hbm.at[col_v.at[c]], gbufs[b], semg[b])
            pltpu.async_copy(ew_hbm.at[sid, c], ewb[b], semw[b])

        def compute(c, b):
            himask = jnp.full((LANES,), -65536, jnp.int32)

            @plsc.parallel_loop(0, CHUNK // LANES)
            def _(g):
                w16 = ewb[b][pl.ds(g * LANES, LANES)]
                for l in range(LANES):
                    w = w16[l]
                    r = g * LANES + l
                    for k in range(dh // (2 * LANES)):
                        v = gbufs[b][r, pl.ds(k * LANES, LANES)]
                        fe = lax.bitcast_convert_type(v << 16, jnp.float32)
                        fo = lax.bitcast_convert_type(v & himask, jnp.float32)
                        sbufs[b][r, pl.ds(k * 2 * LANES, LANES)] = fe * w
                        sbufs[b][r, pl.ds(k * 2 * LANES + LANES, LANES)] = fo * w

        def issue_scatter(c, b):
            pltpu.async_copy(sbufs[b], acc_sh.at[row_v.at[c]], sems[b], add=True)

        def wait_gather(b):
            pltpu.make_async_copy(x_hbm.at[col_v.at[0]], gbufs[b], semg[b]).wait()
            pltpu.make_async_copy(ew_hbm.at[sid, 0], ewb[b], semw[b]).wait()

        def wait_scatter(b, c):
            pltpu.make_async_copy(sbufs[b], acc_sh.at[row_v.at[c]], sems[b]).wait()

        for b in range(NBUF):
            issue_gather(b, b)
            if b >= GLAG:
                bc = b - GLAG
                wait_gather(bc)
                compute(bc, bc)
                issue_scatter(bc, bc)

        def group_body(m, carry):
            g = m * NBUF
            for b in range(NBUF):
                c = g + b
                wait_scatter(b, c)
                issue_gather(c, b)
                bc = (b - GLAG) % NBUF
                wait_gather(bc)
                compute(c - GLAG, bc)
                issue_scatter(c - GLAG, bc)
            return carry

        lax.fori_loop(1, n_chunks // NBUF, group_body, 0)

        for i in range(GLAG):
            c = n_chunks - GLAG + i
            b = c % NBUF
            wait_gather(b)
            compute(c, b)
            issue_scatter(c, b)
        for i in range(NBUF):
            c = n_chunks - NBUF + i
            wait_scatter(c % NBUF, c)
        plsc.subcore_barrier()

        for t in range(zrounds):
            blk_id = sid + t * NS

            @pl.when(blk_id < n_zblk)
            def _():
                pltpu.sync_copy(acc_sh.at[pl.ds(blk_id * CHUNK, CHUNK)],
                                out_hbm.at[cid, pl.ds(blk_id * CHUNK, CHUNK)])

    return spmm_kernel(x2, row3, col3, ew3)


def _tc_combine_matmul_relu(parts, W):
    n_total, dh = parts.shape[1], parts.shape[2]
    d_in, d_out = W.shape
    blk = 2000
    grid = n_total // blk

    def body(p_ref, w_ref, o_ref):
        s = jnp.concatenate([p_ref[0], p_ref[1]], axis=1)
        o_ref[...] = jnp.maximum(
            jnp.dot(s, w_ref[...], preferred_element_type=jnp.float32), 0.0
        )

    return pl.pallas_call(
        body,
        grid=(grid,),
        in_specs=[
            pl.BlockSpec((NC, blk, dh), lambda i: (0, i, 0)),
            pl.BlockSpec((d_in, d_out), lambda i: (0, 0)),
        ],
        out_specs=pl.BlockSpec((blk, d_out), lambda i: (i, 0)),
        out_shape=jax.ShapeDtypeStruct((n_total, d_out), jnp.float32),
    )(parts, W)


def kernel(x, edge_index, edge_weight, W):
    n_edges = edge_index.shape[1]
    dh = x.shape[1] // NC
    n_chunks = n_edges // (NS * CHUNK)
    xb = x.astype(jnp.bfloat16).reshape(NC * x.shape[0], dh // 2, 2)
    x2 = jax.lax.bitcast_convert_type(xb, jnp.int32)
    row3 = edge_index[0].reshape(NS, n_chunks, CHUNK)
    col3 = edge_index[1].reshape(NS, n_chunks, CHUNK)
    ew3 = edge_weight.reshape(NS, n_chunks, CHUNK)
    parts = _sc_spmm(x2, row3, col3, ew3, dh)
    qperm = []
    for h in range(NC):
        for kb in range(dh // 32):
            base = h * dh + kb * 32
            qperm += [base + 2 * j for j in range(16)]
            qperm += [base + 2 * j + 1 for j in range(16)]
    Wp = W[jnp.array(qperm, dtype=jnp.int32), :]
    return _tc_combine_matmul_relu(parts, Wp)


# speedup vs baseline: 5.8979x; 5.8979x over previous
import functools

import jax
import jax.numpy as jnp
from jax import lax
from jax.experimental import pallas as pl
from jax.experimental.pallas import tpu as pltpu
from jax.experimental.pallas import tpu_sc as plsc

NC = 2
NS = 16
NW = NC * NS
CHUNK = 40
LANES = 16
NBUF = 5
GLAG = 2


def _sc_spmm(x, row3, col3, ew3):
    n_nodes, dh = x.shape
    n_chunks = row3.shape[1]
    dh_vecs = dh // LANES
    n_zblk = n_nodes // CHUNK
    zrounds = -(-n_zblk // NS)

    mesh = plsc.VectorSubcoreMesh(
        core_axis_name="c", subcore_axis_name="s", num_cores=NC, num_subcores=NS
    )

    @functools.partial(
        pl.kernel,
        out_type=jax.ShapeDtypeStruct((NC, n_nodes, dh), jnp.float32),
        mesh=mesh,
        compiler_params=pltpu.CompilerParams(use_tc_tiling_on_sc=False),
        scratch_types=[
            pltpu.VMEM((n_chunks, CHUNK), jnp.int32),
            pltpu.VMEM((n_chunks, CHUNK), jnp.int32),
            [pltpu.VMEM((64,), jnp.float32) for _ in range(NBUF)],
            [pltpu.VMEM((CHUNK, dh), jnp.float32) for _ in range(NBUF)],
            [pltpu.SemaphoreType.DMA for _ in range(NBUF)],
            [pltpu.SemaphoreType.DMA for _ in range(NBUF)],
            [pltpu.SemaphoreType.DMA for _ in range(NBUF)],
            pltpu.VMEM_SHARED((n_nodes, dh), jnp.float32),
        ],
    )
    def spmm_kernel(x_hbm, row_hbm, col_hbm, ew_hbm, out_hbm,
                    col_v, row_v, ewb, bufs, semg, semw, sems, acc_sh):
        cid = lax.axis_index("c")
        sid = lax.axis_index("s")
        wid = sid * NC + cid

        pltpu.sync_copy(col_hbm.at[wid], col_v)
        pltpu.sync_copy(row_hbm.at[wid], row_v)

        def zero_row(i, carry):
            for k in range(dh_vecs):
                bufs[0][i, pl.ds(k * LANES, LANES)] = jnp.zeros((LANES,), jnp.float32)
            return carry

        lax.fori_loop(0, CHUNK, zero_row, 0)
        for t in range(zrounds):
            blk_id = sid + t * NS

            @pl.when(blk_id < n_zblk)
            def _():
                pltpu.sync_copy(bufs[0], acc_sh.at[pl.ds(blk_id * CHUNK, CHUNK)])
        plsc.subcore_barrier()

        def issue_gather(c, b):
            pltpu.async_copy(x_hbm.at[col_v.at[c]], bufs[b], semg[b])
            pltpu.async_copy(ew_hbm.at[wid, c], ewb[b].at[pl.ds(0, CHUNK)], semw[b])

        def compute(c, b):
            @plsc.parallel_loop(0, CHUNK // LANES)
            def _(g):
                w16 = ewb[b][pl.ds(g * LANES, LANES)]
                for l in range(LANES):
                    w = w16[l]
                    for k in range(dh_vecs):
                        sl = pl.ds(k * LANES, LANES)
                        bufs[b][g * LANES + l, sl] = bufs[b][g * LANES + l, sl] * w
            tail = CHUNK - (CHUNK // LANES) * LANES
            if tail:
                tbase = (CHUNK // LANES) * LANES
                w16 = ewb[b][pl.ds(tbase, LANES)]
                for l in range(tail):
                    w = w16[l]
                    for k in range(dh_vecs):
                        sl = pl.ds(k * LANES, LANES)
                        bufs[b][tbase + l, sl] = bufs[b][tbase + l, sl] * w

        def issue_scatter(c, b):
            pltpu.async_copy(bufs[b], acc_sh.at[row_v.at[c]], sems[b], add=True)

        def wait_gather(b):
            pltpu.make_async_copy(x_hbm.at[col_v.at[0]], bufs[b], semg[b]).wait()
            pltpu.make_async_copy(ew_hbm.at[wid, 0], ewb[b].at[pl.ds(0, CHUNK)], semw[b]).wait()

        def wait_scatter(b, c):
            pltpu.make_async_copy(bufs[b], acc_sh.at[row_v.at[c]], sems[b]).wait()

        for b in range(NBUF):
            issue_gather(b, b)
            if b >= GLAG:
                bc = b - GLAG
                wait_gather(bc)
                compute(bc, bc)
                issue_scatter(bc, bc)

        def group_body(m, carry):
            g = m * NBUF
            for b in range(NBUF):
                c = g + b
                wait_scatter(b, c)
                issue_gather(c, b)
                bc = (b - GLAG) % NBUF
                wait_gather(bc)
                compute(c - GLAG, bc)
                issue_scatter(c - GLAG, bc)
            return carry

        lax.fori_loop(1, n_chunks // NBUF, group_body, 0)

        for i in range(GLAG):
            c = n_chunks - GLAG + i
            b = c % NBUF
            wait_gather(b)
            compute(c, b)
            issue_scatter(c, b)
        for i in range(NBUF):
            c = n_chunks - NBUF + i
            wait_scatter(c % NBUF, c)
        plsc.subcore_barrier()

        for t in range(zrounds):
            blk_id = sid + t * NS

            @pl.when(blk_id < n_zblk)
            def _():
                pltpu.sync_copy(acc_sh.at[pl.ds(blk_id * CHUNK, CHUNK)],
                                out_hbm.at[cid, pl.ds(blk_id * CHUNK, CHUNK)])

    return spmm_kernel(x, row3, col3, ew3)


def _tc_combine_matmul_relu(parts, W):
    n_total, dh = parts.shape[1], parts.shape[2]
    d_in, d_out = W.shape
    blk = 2000
    grid = n_total // blk

    def body(p_ref, w_ref, o_ref):
        s = p_ref[0] + p_ref[1]
        o_ref[...] = jnp.maximum(
            jnp.dot(s, w_ref[...], preferred_element_type=jnp.float32), 0.0
        )

    return pl.pallas_call(
        body,
        grid=(grid,),
        in_specs=[
            pl.BlockSpec((NC, blk, dh), lambda i: (0, i, 0)),
            pl.BlockSpec((d_in, d_out), lambda i: (0, 0)),
        ],
        out_specs=pl.BlockSpec((blk, d_out), lambda i: (i, 0)),
        out_shape=jax.ShapeDtypeStruct((n_total, d_out), jnp.float32),
    )(parts, W)


def kernel(x, edge_index, edge_weight, W):
    n_edges = edge_index.shape[1]
    n_chunks = n_edges // (NW * CHUNK)
    row3 = edge_index[0].reshape(NW, n_chunks, CHUNK)
    col3 = edge_index[1].reshape(NW, n_chunks, CHUNK)
    ew3 = edge_weight.reshape(NW, n_chunks, CHUNK)
    parts = _sc_spmm(x, row3, col3, ew3)
    return _tc_combine_matmul_relu(parts, W)
